# baseline (speedup 1.0000x reference)
import functools

import jax
import jax.numpy as jnp
from jax import lax
from jax.experimental import pallas as pl
from jax.experimental.pallas import tpu as pltpu
from jax.experimental.pallas import tpu_sc as plsc

_N = 10000
_E = 320000
_D_IN = 128
_D_HID = 128
_D_OUT = 16

_NC = 2
_NS = 16
_NBUF = 4

_PER_TILE = _E // (_NC * _NS)
_KW = _D_IN // _NC
_PER_TILE1 = _E // _NS
_K1 = 50
_STEPS1 = _PER_TILE1 // _K1
_TROWS = _N // _NS
_K2 = 125
_STEPS2 = _PER_TILE // _K2
_NP = 10240
_ROWS = _NP // _NS

_BLK = 2000
_GRID = _N // _BLK


def _zero_fill(buf, nrow, ncol):
    zv = jnp.zeros((16,), jnp.float32)

    def row(i, carry):
        for j in range(ncol // 16):
            buf[i, pl.ds(16 * j, 16)] = zv
        return carry

    lax.fori_loop(0, nrow, row, 0)


def _zero_shared(buf, zsrc, r0, nrows, kz):
    nfull = nrows // kz
    rem = nrows - nfull * kz
    for q in range(nfull):
        pltpu.sync_copy(zsrc, buf.at[pl.ds(r0 + q * kz, kz)])
    if rem:
        pltpu.sync_copy(zsrc.at[pl.ds(0, rem)], buf.at[pl.ds(r0 + nfull * kz, rem)])


def _sc_segment_sum_count(x2, src, dst):
    mesh = plsc.VectorSubcoreMesh(core_axis_name="c", subcore_axis_name="s")

    @functools.partial(
        pl.kernel,
        out_type=[
            jax.ShapeDtypeStruct((_NC, _NP, _KW), jnp.float32),
            jax.ShapeDtypeStruct((_NC, _NP, 16), jnp.float32),
        ],
        mesh=mesh,
        compiler_params=pltpu.CompilerParams(use_tc_tiling_on_sc=False),
        scratch_types=[
            pltpu.VMEM((_STEPS1, _K1), jnp.int32),
            [pltpu.VMEM((_K1,), jnp.int32) for _ in range(_NBUF)],
            [pltpu.VMEM((_K1, _KW), jnp.float32) for _ in range(_NBUF)],
            pltpu.VMEM((_K1, 16), jnp.float32),
            pltpu.VMEM_SHARED((_N, _KW), jnp.float32),
            pltpu.VMEM_SHARED((_NP, _KW), jnp.float32),
            pltpu.VMEM_SHARED((_NP, 16), jnp.float32),
            [pltpu.SemaphoreType.DMA for _ in range(_NBUF)],
            [pltpu.SemaphoreType.DMA for _ in range(_NBUF)],
            [pltpu.SemaphoreType.DMA for _ in range(_NBUF)],
            [pltpu.SemaphoreType.DMA for _ in range(_NBUF)],
        ],
    )
    def body(x2_h, src_h, dst_h, zc_h, ones_h,
             acc_out_h, cnt_out_h,
             didx, sib, rb, ones_v, tab_sh, acc_sh, cnt_sh,
             semg, semi, sems, semc):
        cid = lax.axis_index("c")
        sid = lax.axis_index("s")
        r0 = sid * _ROWS
        t0 = sid * _TROWS
        pltpu.sync_copy(dst_h.at[sid], didx)
        for b in range(_NBUF):
            pltpu.async_copy(src_h.at[sid, b], sib[b], semi[b])
        pltpu.sync_copy(x2_h.at[cid, pl.ds(t0, _TROWS)],
                        tab_sh.at[pl.ds(t0, _TROWS)])
        pltpu.sync_copy(zc_h, cnt_sh.at[pl.ds(r0, _ROWS)])
        pltpu.sync_copy(ones_h, ones_v)
        _zero_fill(rb[0], _K1, _KW)
        _zero_shared(acc_sh, rb[0], r0, _ROWS, _K1)
        plsc.subcore_barrier()
        for b in range(_NBUF):
            pltpu.make_async_copy(src_h.at[sid, b], sib[b], semi[b]).wait()
            pltpu.async_copy(tab_sh.at[sib[b]], rb[b], semg[b])

        def quad(q, carry):
            c0 = _NBUF * q
            for b in range(_NBUF):
                pltpu.make_async_copy(tab_sh.at[sib[b]], rb[b], semg[b]).wait()
                pltpu.async_copy(rb[b], acc_sh.at[didx.at[c0 + b]], sems[b],
                                 add=True)
                pltpu.async_copy(ones_v, cnt_sh.at[didx.at[c0 + b]], semc[b],
                                 add=True)

                @pl.when(c0 + b + _NBUF < _STEPS1)
                def _():
                    pltpu.async_copy(src_h.at[sid, c0 + b + _NBUF], sib[b],
                                     semi[b])

            for b in range(_NBUF):
                pltpu.make_async_copy(rb[b], acc_sh.at[didx.at[c0 + b]],
                                      sems[b]).wait()
                pltpu.make_async_copy(ones_v, cnt_sh.at[didx.at[c0 + b]],
                                      semc[b]).wait()

                @pl.when(c0 + b + _NBUF < _STEPS1)
                def _():
                    pltpu.make_async_copy(src_h.at[sid, c0 + b + _NBUF],
                                          sib[b], semi[b]).wait()
                    pltpu.async_copy(tab_sh.at[sib[b]], rb[b], semg[b])

            return carry

        lax.fori_loop(0, _STEPS1 // _NBUF, quad, 0)
        plsc.subcore_barrier()
        pltpu.sync_copy(acc_sh.at[pl.ds(r0, _ROWS)],
                        acc_out_h.at[cid, pl.ds(r0, _ROWS)])
        pltpu.sync_copy(cnt_sh.at[pl.ds(r0, _ROWS)],
                        cnt_out_h.at[cid, pl.ds(r0, _ROWS)])

    zc = jnp.zeros((_ROWS, 16), jnp.float32)
    ones = jnp.ones((_K1, 16), jnp.float32)
    return body(x2, src, dst, zc, ones)


def _sc_segment_sum(table, src, dst):
    width = table.shape[1]
    mesh = plsc.VectorSubcoreMesh(core_axis_name="c", subcore_axis_name="s")

    @functools.partial(
        pl.kernel,
        out_type=jax.ShapeDtypeStruct((_NC, _NP, width), jnp.float32),
        mesh=mesh,
        compiler_params=pltpu.CompilerParams(use_tc_tiling_on_sc=False),
        scratch_types=[
            pltpu.VMEM((_STEPS2, _K2), jnp.int32),
            pltpu.VMEM((_STEPS2, _K2), jnp.int32),
            [pltpu.VMEM((_K2, width), jnp.float32) for _ in range(_NBUF)],
            pltpu.VMEM_SHARED((_NP, width), jnp.float32),
            [pltpu.SemaphoreType.DMA for _ in range(_NBUF)],
            [pltpu.SemaphoreType.DMA for _ in range(_NBUF)],
        ],
    )
    def body(table_h, src_h, dst_h, acc_out_h,
             sidx, didx, rb, acc_sh, semg, sems):
        cid = lax.axis_index("c")
        sid = lax.axis_index("s")
        wid = cid * _NS + sid
        r0 = sid * _ROWS
        pltpu.sync_copy(src_h.at[wid], sidx)
        pltpu.sync_copy(dst_h.at[wid], didx)
        _zero_fill(rb[0], _K2, width)
        _zero_shared(acc_sh, rb[0], r0, _ROWS, _K2)
        plsc.subcore_barrier()
        for b in range(_NBUF):
            pltpu.async_copy(table_h.at[sidx.at[b]], rb[b], semg[b])

        def quad(q, carry):
            c0 = _NBUF * q
            for b in range(_NBUF):
                pltpu.make_async_copy(table_h.at[sidx.at[c0 + b]], rb[b],
                                      semg[b]).wait()
                pltpu.async_copy(rb[b], acc_sh.at[didx.at[c0 + b]], sems[b],
                                 add=True)
            for b in range(_NBUF):
                pltpu.make_async_copy(rb[b], acc_sh.at[didx.at[c0 + b]],
                                      sems[b]).wait()

                @pl.when(c0 + b + _NBUF < _STEPS2)
                def _():
                    pltpu.async_copy(table_h.at[sidx.at[c0 + b + _NBUF]],
                                     rb[b], semg[b])

            return carry

        lax.fori_loop(0, _STEPS2 // _NBUF, quad, 0)
        plsc.subcore_barrier()
        pltpu.sync_copy(acc_sh.at[pl.ds(r0, _ROWS)],
                        acc_out_h.at[cid, pl.ds(r0, _ROWS)])

    return body(table, src, dst)


def _tc_r1(x, W1_r, b1):
    def body(x_ref, wr_ref, b_ref, r1_ref):
        r1_ref[...] = jnp.dot(x_ref[...], wr_ref[...],
                              preferred_element_type=jnp.float32) + b_ref[...]

    return pl.pallas_call(
        body,
        grid=(_GRID,),
        in_specs=[
            pl.BlockSpec((_BLK, _D_IN), lambda i: (i, 0)),
            pl.BlockSpec((_D_IN, _D_HID), lambda i: (0, 0)),
            pl.BlockSpec((1, _D_HID), lambda i: (0, 0)),
        ],
        out_specs=pl.BlockSpec((_BLK, _D_HID), lambda i: (i, 0)),
        out_shape=jax.ShapeDtypeStruct((_N, _D_HID), jnp.float32),
    )(x, W1_r, b1.reshape(1, _D_HID))


def _tc_middle(p1, c1, r1, W1_l, W2_l, W2_r, b2):
    def body(p_ref, c_ref, r_ref, w1_ref, wl_ref, wr_ref, b_ref,
             y2_ref, r2_ref):
        acc = jnp.concatenate([p_ref[0], p_ref[1]], axis=-1)
        cnt = c_ref[0, :, 0:1]
        mean = acc / jnp.maximum(cnt, 1.0)
        h = jnp.maximum(
            jnp.dot(mean, w1_ref[...], preferred_element_type=jnp.float32)
            + r_ref[...], 0.0)
        y2_ref[...] = jnp.dot(h, wl_ref[...],
                              preferred_element_type=jnp.float32)
        r2_ref[...] = jnp.dot(h, wr_ref[...],
                              preferred_element_type=jnp.float32) + b_ref[...]

    return pl.pallas_call(
        body,
        grid=(_GRID,),
        in_specs=[
            pl.BlockSpec((_NC, _BLK, _KW), lambda i: (0, i, 0)),
            pl.BlockSpec((_NC, _BLK, 16), lambda i: (0, i, 0)),
            pl.BlockSpec((_BLK, _D_HID), lambda i: (i, 0)),
            pl.BlockSpec((_D_IN, _D_HID), lambda i: (0, 0)),
            pl.BlockSpec((_D_HID, _D_OUT), lambda i: (0, 0)),
            pl.BlockSpec((_D_HID, _D_OUT), lambda i: (0, 0)),
            pl.BlockSpec((1, _D_OUT), lambda i: (0, 0)),
        ],
        out_specs=[
            pl.BlockSpec((_BLK, _D_OUT), lambda i: (i, 0)),
            pl.BlockSpec((_BLK, _D_OUT), lambda i: (i, 0)),
        ],
        out_shape=[
            jax.ShapeDtypeStruct((_N, _D_OUT), jnp.float32),
            jax.ShapeDtypeStruct((_N, _D_OUT), jnp.float32),
        ],
    )(p1, c1, r1, W1_l, W2_l, W2_r, b2.reshape(1, _D_OUT))


def _tc_final(p2, c1, r2):
    def body(p_ref, c_ref, r_ref, o_ref):
        acc = p_ref[0] + p_ref[1]
        cnt = c_ref[0, :, 0:1]
        o_ref[...] = acc / jnp.maximum(cnt, 1.0) + r_ref[...]

    return pl.pallas_call(
        body,
        grid=(_GRID,),
        in_specs=[
            pl.BlockSpec((_NC, _BLK, _D_OUT), lambda i: (0, i, 0)),
            pl.BlockSpec((_NC, _BLK, 16), lambda i: (0, i, 0)),
            pl.BlockSpec((_BLK, _D_OUT), lambda i: (i, 0)),
        ],
        out_specs=pl.BlockSpec((_BLK, _D_OUT), lambda i: (i, 0)),
        out_shape=jax.ShapeDtypeStruct((_N, _D_OUT), jnp.float32),
    )(p2, c1, r2)


def kernel(x, last_update, edge_index, t, msg, W1_l, W1_r, b1, W2_l, W2_r, b2):
    src = edge_index[0]
    dst = edge_index[1]
    src1 = src.reshape(_NS, _STEPS1, _K1)
    dst1 = dst.reshape(_NS, _STEPS1, _K1)
    src2 = src.reshape(_NC * _NS, _STEPS2, _K2)
    dst2 = dst.reshape(_NC * _NS, _STEPS2, _K2)
    x2 = jnp.stack([x[:, :_KW], x[:, _KW:]])

    p1, c1 = _sc_segment_sum_count(x2, src1, dst1)
    r1 = _tc_r1(x, W1_r, b1)
    y2, r2 = _tc_middle(p1, c1, r1, W1_l, W2_l, W2_r, b2)
    p2 = _sc_segment_sum(y2, src2, dst2)
    return _tc_final(p2, c1, r2)


# speedup vs baseline: 1.3797x; 1.3797x over previous
import functools

import jax
import jax.numpy as jnp
from jax import lax
from jax.experimental import pallas as pl
from jax.experimental.pallas import tpu as pltpu
from jax.experimental.pallas import tpu_sc as plsc

_N = 10000
_E = 320000
_D_IN = 128
_D_HID = 128
_D_OUT = 16

_NC = 2
_NS = 16
_NBUF = 4

_PER_TILE = _E // (_NC * _NS)
_K1 = 50
_STEPS1 = _PER_TILE // _K1
_K2 = 125
_STEPS2 = _PER_TILE // _K2
_NP = 10240
_ROWS = _NP // _NS

_BLK = 2000
_GRID = _N // _BLK


def _zero_fill(buf, nrow, ncol):
    zv = jnp.zeros((16,), jnp.float32)

    def row(i, carry):
        for j in range(ncol // 16):
            buf[i, pl.ds(16 * j, 16)] = zv
        return carry

    lax.fori_loop(0, nrow, row, 0)


def _zero_shared(buf, zsrc, r0, nrows, kz):
    nfull = nrows // kz
    rem = nrows - nfull * kz
    for q in range(nfull):
        pltpu.sync_copy(zsrc, buf.at[pl.ds(r0 + q * kz, kz)])
    if rem:
        pltpu.sync_copy(zsrc.at[pl.ds(0, rem)], buf.at[pl.ds(r0 + nfull * kz, rem)])


def _sc_segment_sum_count(table, src, dst):
    width = table.shape[1]
    mesh = plsc.VectorSubcoreMesh(core_axis_name="c", subcore_axis_name="s")

    @functools.partial(
        pl.kernel,
        out_type=[
            jax.ShapeDtypeStruct((_NC, _NP, width), jnp.float32),
            jax.ShapeDtypeStruct((_NC, _NP, 16), jnp.float32),
        ],
        mesh=mesh,
        compiler_params=pltpu.CompilerParams(use_tc_tiling_on_sc=False),
        scratch_types=[
            pltpu.VMEM((_STEPS1, _K1), jnp.int32),
            [pltpu.VMEM((_K1,), jnp.int32) for _ in range(_NBUF)],
            [pltpu.VMEM((_K1, width), jnp.float32) for _ in range(_NBUF)],
            pltpu.VMEM((_K1, 16), jnp.float32),
            pltpu.VMEM_SHARED((_NP, width), jnp.float32),
            pltpu.VMEM_SHARED((_NP, 16), jnp.float32),
            [pltpu.SemaphoreType.DMA for _ in range(_NBUF)],
            [pltpu.SemaphoreType.DMA for _ in range(_NBUF)],
            [pltpu.SemaphoreType.DMA for _ in range(_NBUF)],
            [pltpu.SemaphoreType.DMA for _ in range(_NBUF)],
        ],
    )
    def body(table_h, src_h, dst_h, zc_h, ones_h,
             acc_out_h, cnt_out_h,
             didx, sib, rb, ones_v, acc_sh, cnt_sh,
             semg, semi, sems, semc):
        cid = lax.axis_index("c")
        sid = lax.axis_index("s")
        wid = cid * _NS + sid
        r0 = sid * _ROWS
        pltpu.sync_copy(dst_h.at[wid], didx)
        for b in range(_NBUF):
            pltpu.async_copy(src_h.at[wid, b], sib[b], semi[b])
        pltpu.sync_copy(zc_h, cnt_sh.at[pl.ds(r0, _ROWS)])
        pltpu.sync_copy(ones_h, ones_v)
        _zero_fill(rb[0], _K1, width)
        _zero_shared(acc_sh, rb[0], r0, _ROWS, _K1)
        plsc.subcore_barrier()
        for b in range(_NBUF):
            pltpu.make_async_copy(src_h.at[wid, b], sib[b], semi[b]).wait()
            pltpu.async_copy(table_h.at[sib[b]], rb[b], semg[b])

        def quad(q, carry):
            c0 = _NBUF * q
            for b in range(_NBUF):
                pltpu.make_async_copy(table_h.at[sib[b]], rb[b], semg[b]).wait()
                pltpu.async_copy(rb[b], acc_sh.at[didx.at[c0 + b]], sems[b],
                                 add=True)
                pltpu.async_copy(ones_v, cnt_sh.at[didx.at[c0 + b]], semc[b],
                                 add=True)

                @pl.when(c0 + b + _NBUF < _STEPS1)
                def _():
                    pltpu.async_copy(src_h.at[wid, c0 + b + _NBUF], sib[b],
                                     semi[b])

            for b in range(_NBUF):
                pltpu.make_async_copy(rb[b], acc_sh.at[didx.at[c0 + b]],
                                      sems[b]).wait()
                pltpu.make_async_copy(ones_v, cnt_sh.at[didx.at[c0 + b]],
                                      semc[b]).wait()

                @pl.when(c0 + b + _NBUF < _STEPS1)
                def _():
                    pltpu.make_async_copy(src_h.at[wid, c0 + b + _NBUF],
                                          sib[b], semi[b]).wait()
                    pltpu.async_copy(table_h.at[sib[b]], rb[b], semg[b])

            return carry

        lax.fori_loop(0, _STEPS1 // _NBUF, quad, 0)
        plsc.subcore_barrier()
        pltpu.sync_copy(acc_sh.at[pl.ds(r0, _ROWS)],
                        acc_out_h.at[cid, pl.ds(r0, _ROWS)])
        pltpu.sync_copy(cnt_sh.at[pl.ds(r0, _ROWS)],
                        cnt_out_h.at[cid, pl.ds(r0, _ROWS)])

    zc = jnp.zeros((_ROWS, 16), jnp.float32)
    ones = jnp.ones((_K1, 16), jnp.float32)
    return body(table, src, dst, zc, ones)


def _sc_segment_sum(table, src, dst):
    width = table.shape[1]
    mesh = plsc.VectorSubcoreMesh(core_axis_name="c", subcore_axis_name="s")

    @functools.partial(
        pl.kernel,
        out_type=jax.ShapeDtypeStruct((_NC, _NP, width), jnp.float32),
        mesh=mesh,
        compiler_params=pltpu.CompilerParams(use_tc_tiling_on_sc=False),
        scratch_types=[
            pltpu.VMEM((_STEPS2, _K2), jnp.int32),
            pltpu.VMEM((_STEPS2, _K2), jnp.int32),
            [pltpu.VMEM((_K2, width), jnp.float32) for _ in range(_NBUF)],
            pltpu.VMEM_SHARED((_NP, width), jnp.float32),
            [pltpu.SemaphoreType.DMA for _ in range(_NBUF)],
            [pltpu.SemaphoreType.DMA for _ in range(_NBUF)],
        ],
    )
    def body(table_h, src_h, dst_h, acc_out_h,
             sidx, didx, rb, acc_sh, semg, sems):
        cid = lax.axis_index("c")
        sid = lax.axis_index("s")
        wid = cid * _NS + sid
        r0 = sid * _ROWS
        pltpu.sync_copy(src_h.at[wid], sidx)
        pltpu.sync_copy(dst_h.at[wid], didx)
        _zero_fill(rb[0], _K2, width)
        _zero_shared(acc_sh, rb[0], r0, _ROWS, _K2)
        plsc.subcore_barrier()
        for b in range(_NBUF):
            pltpu.async_copy(table_h.at[sidx.at[b]], rb[b], semg[b])

        def quad(q, carry):
            c0 = _NBUF * q
            for b in range(_NBUF):
                pltpu.make_async_copy(table_h.at[sidx.at[c0 + b]], rb[b],
                                      semg[b]).wait()
                pltpu.async_copy(rb[b], acc_sh.at[didx.at[c0 + b]], sems[b],
                                 add=True)
            for b in range(_NBUF):
                pltpu.make_async_copy(rb[b], acc_sh.at[didx.at[c0 + b]],
                                      sems[b]).wait()

                @pl.when(c0 + b + _NBUF < _STEPS2)
                def _():
                    pltpu.async_copy(table_h.at[sidx.at[c0 + b + _NBUF]],
                                     rb[b], semg[b])

            return carry

        lax.fori_loop(0, _STEPS2 // _NBUF, quad, 0)
        plsc.subcore_barrier()
        pltpu.sync_copy(acc_sh.at[pl.ds(r0, _ROWS)],
                        acc_out_h.at[cid, pl.ds(r0, _ROWS)])

    return body(table, src, dst)


def _tc_middle(p1, c1, x, W1_l, W1_r, b1, W2_l, W2_r, b2):
    def body(p_ref, c_ref, x_ref, w1l_ref, w1r_ref, b1_ref,
             wl_ref, wr_ref, b2_ref, y2_ref, r2_ref):
        acc = p_ref[0] + p_ref[1]
        cnt = c_ref[0, :, 0:1] + c_ref[1, :, 0:1]
        mean = acc / jnp.maximum(cnt, 1.0)
        h = jnp.maximum(
            jnp.dot(mean, w1l_ref[...], preferred_element_type=jnp.float32)
            + jnp.dot(x_ref[...], w1r_ref[...],
                      preferred_element_type=jnp.float32)
            + b1_ref[...], 0.0)
        y2_ref[...] = jnp.dot(h, wl_ref[...],
                              preferred_element_type=jnp.float32)
        r2_ref[...] = jnp.dot(h, wr_ref[...],
                              preferred_element_type=jnp.float32) + b2_ref[...]

    return pl.pallas_call(
        body,
        grid=(_GRID,),
        in_specs=[
            pl.BlockSpec((_NC, _BLK, _D_IN), lambda i: (0, i, 0)),
            pl.BlockSpec((_NC, _BLK, 16), lambda i: (0, i, 0)),
            pl.BlockSpec((_BLK, _D_IN), lambda i: (i, 0)),
            pl.BlockSpec((_D_IN, _D_HID), lambda i: (0, 0)),
            pl.BlockSpec((_D_IN, _D_HID), lambda i: (0, 0)),
            pl.BlockSpec((1, _D_HID), lambda i: (0, 0)),
            pl.BlockSpec((_D_HID, _D_OUT), lambda i: (0, 0)),
            pl.BlockSpec((_D_HID, _D_OUT), lambda i: (0, 0)),
            pl.BlockSpec((1, _D_OUT), lambda i: (0, 0)),
        ],
        out_specs=[
            pl.BlockSpec((_BLK, _D_OUT), lambda i: (i, 0)),
            pl.BlockSpec((_BLK, _D_OUT), lambda i: (i, 0)),
        ],
        out_shape=[
            jax.ShapeDtypeStruct((_N, _D_OUT), jnp.float32),
            jax.ShapeDtypeStruct((_N, _D_OUT), jnp.float32),
        ],
    )(p1, c1, x, W1_l, W1_r, b1.reshape(1, _D_HID),
      W2_l, W2_r, b2.reshape(1, _D_OUT))


def _tc_final(p2, c1, r2):
    def body(p_ref, c_ref, r_ref, o_ref):
        acc = p_ref[0] + p_ref[1]
        cnt = c_ref[0, :, 0:1] + c_ref[1, :, 0:1]
        o_ref[...] = acc / jnp.maximum(cnt, 1.0) + r_ref[...]

    return pl.pallas_call(
        body,
        grid=(_GRID,),
        in_specs=[
            pl.BlockSpec((_NC, _BLK, _D_OUT), lambda i: (0, i, 0)),
            pl.BlockSpec((_NC, _BLK, 16), lambda i: (0, i, 0)),
            pl.BlockSpec((_BLK, _D_OUT), lambda i: (i, 0)),
        ],
        out_specs=pl.BlockSpec((_BLK, _D_OUT), lambda i: (i, 0)),
        out_shape=jax.ShapeDtypeStruct((_N, _D_OUT), jnp.float32),
    )(p2, c1, r2)


def kernel(x, last_update, edge_index, t, msg, W1_l, W1_r, b1, W2_l, W2_r, b2):
    src = edge_index[0]
    dst = edge_index[1]
    src1 = src.reshape(_NC * _NS, _STEPS1, _K1)
    dst1 = dst.reshape(_NC * _NS, _STEPS1, _K1)
    src2 = src.reshape(_NC * _NS, _STEPS2, _K2)
    dst2 = dst.reshape(_NC * _NS, _STEPS2, _K2)

    p1, c1 = _sc_segment_sum_count(x, src1, dst1)
    y2, r2 = _tc_middle(p1, c1, x, W1_l, W1_r, b1, W2_l, W2_r, b2)
    p2 = _sc_segment_sum(y2, src2, dst2)
    return _tc_final(p2, c1, r2)
